# baseline (speedup 1.0000x reference)
import functools
import math

import jax
import jax.numpy as jnp
from jax import lax
from jax.experimental import pallas as pl
from jax.experimental.pallas import tpu as pltpu
from jax.experimental.pallas import tpu_sc as plsc

_VOCAB = 100000
_DIM = 128
_SCALE = math.sqrt(_DIM)

_NC = 2
_NS = 16
_NW = _NC * _NS

_SPAD = 64
_ROWS = 4
_NBUF = 4


@functools.partial(jax.jit, static_argnames=("n", "s"))
def _lookup(idx, table, *, n, s):
    rows_per_w = n // _NW
    n_chunks = rows_per_w // _ROWS
    mesh = plsc.VectorSubcoreMesh(core_axis_name="c", subcore_axis_name="s")

    @functools.partial(
        pl.kernel,
        mesh=mesh,
        out_type=jax.ShapeDtypeStruct((n, s, _DIM), jnp.float32),
        scratch_types=[
            pltpu.VMEM((rows_per_w * _SPAD,), jnp.int32),
            pltpu.VMEM((_NBUF, _ROWS, s, _DIM), jnp.float32),
            pltpu.SemaphoreType.DMA,
            pltpu.SemaphoreType.DMA,
            pltpu.SemaphoreType.DMA,
            pltpu.SemaphoreType.DMA,
            pltpu.SemaphoreType.DMA,
            pltpu.SemaphoreType.DMA,
            pltpu.SemaphoreType.DMA,
            pltpu.SemaphoreType.DMA,
        ],
    )
    def k(idx_hbm, table_hbm, out_hbm, idx_v, rows_v, g0, g1, g2, g3, w0, w1, w2, w3):
        gsem = [g0, g1, g2, g3]
        wsem = [w0, w1, w2, w3]
        wid = lax.axis_index("s") * _NC + lax.axis_index("c")
        row_base = wid * rows_per_w

        pltpu.sync_copy(
            idx_hbm.at[pl.ds(row_base * _SPAD, rows_per_w * _SPAD)],
            idx_v.at[...],
        )

        def fire_gather(c):
            b = c % _NBUF
            return [
                pltpu.async_copy(
                    table_hbm.at[idx_v.at[pl.ds((c * _ROWS + r) * _SPAD, s)]],
                    rows_v.at[b, r],
                    gsem[b],
                )
                for r in range(_ROWS)
            ]

        def scale(b):
            def scale_row(i, carry):
                for j in range(_DIM // 16):
                    sl = (b, i // s, i % s, pl.ds(j * 16, 16))
                    rows_v[sl] = rows_v[sl] * _SCALE
                return carry

            lax.fori_loop(0, _ROWS * s, scale_row, 0)

        pending = {c: fire_gather(c) for c in range(min(3, n_chunks))}
        wb = {}
        for c in range(n_chunks):
            b = c % _NBUF
            for cp in pending.pop(c):
                cp.wait()
            scale(b)
            wb[c] = pltpu.async_copy(
                rows_v.at[b],
                out_hbm.at[pl.ds(row_base + c * _ROWS, _ROWS)],
                wsem[b],
            )
            nxt = c + 3
            if nxt < n_chunks:
                prev = c - 1
                if prev in wb:
                    wb.pop(prev).wait()
                pending[nxt] = fire_gather(nxt)
        for c in sorted(wb):
            wb.pop(c).wait()

    return k(idx, table)


def kernel(x, table):
    n, s = x.shape
    idx = jnp.pad(x.astype(jnp.int32), ((0, 0), (0, _SPAD - s))).reshape(n * _SPAD)
    return _lookup(idx, table, n=n, s=s)


# speedup vs baseline: 1.0022x; 1.0022x over previous
import functools
import math

import jax
import jax.numpy as jnp
from jax import lax
from jax.experimental import pallas as pl
from jax.experimental.pallas import tpu as pltpu
from jax.experimental.pallas import tpu_sc as plsc

_VOCAB = 100000
_DIM = 128
_SCALE = math.sqrt(_DIM)

_NC = 2
_NS = 16
_NW = _NC * _NS

_SPAD = 64
_ROWS = 4
_NBUF = 3


@functools.partial(jax.jit, static_argnames=("n", "s"))
def _lookup(idx, table, *, n, s):
    rows_per_w = n // _NW
    n_chunks = rows_per_w // _ROWS
    mesh = plsc.VectorSubcoreMesh(core_axis_name="c", subcore_axis_name="s")

    @functools.partial(
        pl.kernel,
        mesh=mesh,
        out_type=jax.ShapeDtypeStruct((n, s, _DIM), jnp.float32),
        scratch_types=[
            pltpu.VMEM((rows_per_w * _SPAD,), jnp.int32),
            pltpu.VMEM((_NBUF, _ROWS, s, _DIM), jnp.float32),
            pltpu.SemaphoreType.DMA,
            pltpu.SemaphoreType.DMA,
            pltpu.SemaphoreType.DMA,
            pltpu.SemaphoreType.DMA,
            pltpu.SemaphoreType.DMA,
            pltpu.SemaphoreType.DMA,
        ],
    )
    def k(idx_hbm, table_hbm, out_hbm, idx_v, rows_v, g0, g1, g2, w0, w1, w2):
        gsem = [g0, g1, g2]
        wsem = [w0, w1, w2]
        wid = lax.axis_index("s") * _NC + lax.axis_index("c")
        row_base = wid * rows_per_w

        pltpu.sync_copy(
            idx_hbm.at[pl.ds(row_base * _SPAD, rows_per_w * _SPAD)],
            idx_v.at[...],
        )

        def fire_gather(c):
            b = c % _NBUF
            return [
                pltpu.async_copy(
                    table_hbm.at[idx_v.at[pl.ds((c * _ROWS + r) * _SPAD, s)]],
                    rows_v.at[b, r],
                    gsem[b],
                )
                for r in range(_ROWS)
            ]

        def scale(b):
            def scale_row(i, carry):
                for j in range(_DIM // 16):
                    sl = (b, i // s, i % s, pl.ds(j * 16, 16))
                    rows_v[sl] = rows_v[sl] * _SCALE
                return carry

            lax.fori_loop(0, _ROWS * s, scale_row, 0)

        pending = {c: fire_gather(c) for c in range(min(2, n_chunks))}
        wb = {}
        for c in range(n_chunks):
            b = c % _NBUF
            for cp in pending.pop(c):
                cp.wait()
            scale(b)
            wb[c] = pltpu.async_copy(
                rows_v.at[b],
                out_hbm.at[pl.ds(row_base + c * _ROWS, _ROWS)],
                wsem[b],
            )
            nxt = c + 2
            if nxt < n_chunks:
                prev = c - 1
                if prev in wb:
                    wb.pop(prev).wait()
                pending[nxt] = fire_gather(nxt)
        for c in sorted(wb):
            wb.pop(c).wait()

    return k(idx, table)


def kernel(x, table):
    n, s = x.shape
    idx = jnp.pad(x.astype(jnp.int32), ((0, 0), (0, _SPAD - s))).reshape(n * _SPAD)
    return _lookup(idx, table, n=n, s=s)
